# Initial kernel scaffold; baseline (speedup 1.0000x reference)
import jax
import jax.numpy as jnp
from jax.experimental import pallas as pl


def kernel(segment_features, lane_table, type_table, length_table, W, b):
    raise NotImplementedError("write your pallas kernel here")


# speedup vs baseline: 1.3876x; 1.3876x over previous
import functools

import jax
import jax.numpy as jnp
from jax import lax
from jax.experimental import pallas as pl
from jax.experimental.pallas import tpu as pltpu
from jax.experimental.pallas import tpu_sc as plsc

BATCH = 16384
D0 = 32
D1 = 32
D2 = 64
HIDDEN = 128

_NC = 2
_NS = 16
_NW = _NC * _NS
_BPW = BATCH // _NW
_CHUNK = 128
_NCHUNK = _BPW // _CHUNK


def _gather_body(idx0_hbm, idx1_hbm, idx2_hbm, lane_hbm, type_hbm, len_hbm,
                 out0, out1, out2, idx0_v, idx1_v, idx2_v, r0, r1, r2, sem):
    wid = lax.axis_index("s") * _NC + lax.axis_index("c")
    base = wid * _BPW
    pltpu.sync_copy(idx0_hbm.at[pl.ds(base, _BPW)], idx0_v)
    pltpu.sync_copy(idx1_hbm.at[pl.ds(base, _BPW)], idx1_v)
    pltpu.sync_copy(idx2_hbm.at[pl.ds(base, _BPW)], idx2_v)
    copies = []
    for c in range(_NCHUNK):
        sl = pl.ds(c * _CHUNK, _CHUNK)
        copies.append(pltpu.async_copy(lane_hbm.at[idx0_v.at[sl]], r0.at[sl], sem))
        copies.append(pltpu.async_copy(type_hbm.at[idx1_v.at[sl]], r1.at[sl], sem))
        copies.append(pltpu.async_copy(len_hbm.at[idx2_v.at[sl]], r2.at[sl], sem))
    for cp in copies:
        cp.wait()
    pltpu.sync_copy(r0, out0.at[pl.ds(base, _BPW)])
    pltpu.sync_copy(r1, out1.at[pl.ds(base, _BPW)])
    pltpu.sync_copy(r2, out2.at[pl.ds(base, _BPW)])


@functools.cache
def _make_gather():
    return pl.kernel(
        _gather_body,
        out_type=[
            jax.ShapeDtypeStruct((BATCH, D0), jnp.float32),
            jax.ShapeDtypeStruct((BATCH, D1), jnp.float32),
            jax.ShapeDtypeStruct((BATCH, D2), jnp.float32),
        ],
        mesh=plsc.VectorSubcoreMesh(core_axis_name="c", subcore_axis_name="s"),
        scratch_types=[
            pltpu.VMEM((_BPW,), jnp.int32),
            pltpu.VMEM((_BPW,), jnp.int32),
            pltpu.VMEM((_BPW,), jnp.int32),
            pltpu.VMEM((_BPW, D0), jnp.float32),
            pltpu.VMEM((_BPW, D1), jnp.float32),
            pltpu.VMEM((_BPW, D2), jnp.float32),
            pltpu.SemaphoreType.DMA,
        ],
        compiler_params=pltpu.CompilerParams(use_tc_tiling_on_sc=False),
    )


_MM_ROWS = 1024


def _mm_body(x0_ref, x1_ref, x2_ref, w0_ref, w1_ref, w2_ref, b_ref, o_ref):
    acc = jnp.dot(x0_ref[...], w0_ref[...], preferred_element_type=jnp.float32)
    acc += jnp.dot(x1_ref[...], w1_ref[...], preferred_element_type=jnp.float32)
    acc += jnp.dot(x2_ref[...], w2_ref[...], preferred_element_type=jnp.float32)
    o_ref[...] = acc + b_ref[...]


_matmul = pl.pallas_call(
    _mm_body,
    grid=(BATCH // _MM_ROWS,),
    in_specs=[
        pl.BlockSpec((_MM_ROWS, D0), lambda i: (i, 0)),
        pl.BlockSpec((_MM_ROWS, D1), lambda i: (i, 0)),
        pl.BlockSpec((_MM_ROWS, D2), lambda i: (i, 0)),
        pl.BlockSpec((D0, HIDDEN), lambda i: (0, 0)),
        pl.BlockSpec((D1, HIDDEN), lambda i: (0, 0)),
        pl.BlockSpec((D2, HIDDEN), lambda i: (0, 0)),
        pl.BlockSpec((1, HIDDEN), lambda i: (0, 0)),
    ],
    out_specs=pl.BlockSpec((_MM_ROWS, HIDDEN), lambda i: (i, 0)),
    out_shape=jax.ShapeDtypeStruct((BATCH, HIDDEN), jnp.float32),
)


@jax.jit
def kernel(segment_features, lane_table, type_table, length_table, W, b):
    idx = segment_features.astype(jnp.int32)
    e0, e1, e2 = _make_gather()(idx[:, 0], idx[:, 1], idx[:, 2],
                                lane_table, type_table, length_table)
    Wt = W.T
    return _matmul(e0, e1, e2, Wt[:D0], Wt[D0:D0 + D1], Wt[D0 + D1:],
                   b.reshape(1, HIDDEN))


# speedup vs baseline: 2.7706x; 1.9967x over previous
import functools

import jax
import jax.numpy as jnp
from jax import lax
from jax.experimental import pallas as pl
from jax.experimental.pallas import tpu as pltpu
from jax.experimental.pallas import tpu_sc as plsc

BATCH = 16384
D0 = 32
D1 = 32
D2 = 64
HIDDEN = 128
V = 100000

_NC = 2
_NS = 16
_NW = _NC * _NS
_RPW = (D0 + D1 + D2) // _NW
_OCHUNK = 4096
_L = 16


def _gather_body(i0, i1, i2, t0T, t1T, t2T, e0T, e1T, e2T,
                 idx_v, row_v, out_v):
    wid = lax.axis_index("s") * _NC + lax.axis_index("c")

    def do_table(tbl, ev, idx_hbm, base_row):
        pltpu.sync_copy(idx_hbm, idx_v)
        for k in range(_RPW):
            c = base_row + k
            pltpu.sync_copy(tbl.at[c], row_v)
            for h in range(BATCH // _OCHUNK):
                def gbody(j, carry):
                    iv = idx_v[pl.ds(h * _OCHUNK + j * _L, _L)]
                    out_v[pl.ds(j * _L, _L)] = plsc.load_gather(row_v, [iv])
                    return carry
                lax.fori_loop(0, _OCHUNK // _L, gbody, 0, unroll=8)
                pltpu.sync_copy(out_v, ev.at[c, pl.ds(h * _OCHUNK, _OCHUNK)])

    @pl.when(wid < 8)
    def _():
        do_table(t0T, e0T, i0, wid * _RPW)

    @pl.when((wid >= 8) & (wid < 16))
    def _():
        do_table(t1T, e1T, i1, (wid - 8) * _RPW)

    @pl.when(wid >= 16)
    def _():
        do_table(t2T, e2T, i2, (wid - 16) * _RPW)


@functools.cache
def _make_gather():
    return pl.kernel(
        _gather_body,
        out_type=[
            jax.ShapeDtypeStruct((D0, BATCH), jnp.float32),
            jax.ShapeDtypeStruct((D1, BATCH), jnp.float32),
            jax.ShapeDtypeStruct((D2, BATCH), jnp.float32),
        ],
        mesh=plsc.VectorSubcoreMesh(core_axis_name="c", subcore_axis_name="s"),
        scratch_types=[
            pltpu.VMEM((BATCH,), jnp.int32),
            pltpu.VMEM((V,), jnp.float32),
            pltpu.VMEM((_OCHUNK,), jnp.float32),
        ],
        compiler_params=pltpu.CompilerParams(needs_layout_passes=False),
    )


_MM_COLS = 1024


def _mm_body(e0_ref, e1_ref, e2_ref, w0_ref, w1_ref, w2_ref, b_ref, o_ref):
    dn = (((0,), (0,)), ((), ()))
    acc = lax.dot_general(e0_ref[...], w0_ref[...], dn,
                          preferred_element_type=jnp.float32)
    acc += lax.dot_general(e1_ref[...], w1_ref[...], dn,
                           preferred_element_type=jnp.float32)
    acc += lax.dot_general(e2_ref[...], w2_ref[...], dn,
                           preferred_element_type=jnp.float32)
    o_ref[...] = acc + b_ref[...]


_matmul = pl.pallas_call(
    _mm_body,
    grid=(BATCH // _MM_COLS,),
    in_specs=[
        pl.BlockSpec((D0, _MM_COLS), lambda i: (0, i)),
        pl.BlockSpec((D1, _MM_COLS), lambda i: (0, i)),
        pl.BlockSpec((D2, _MM_COLS), lambda i: (0, i)),
        pl.BlockSpec((D0, HIDDEN), lambda i: (0, 0)),
        pl.BlockSpec((D1, HIDDEN), lambda i: (0, 0)),
        pl.BlockSpec((D2, HIDDEN), lambda i: (0, 0)),
        pl.BlockSpec((1, HIDDEN), lambda i: (0, 0)),
    ],
    out_specs=pl.BlockSpec((_MM_COLS, HIDDEN), lambda i: (i, 0)),
    out_shape=jax.ShapeDtypeStruct((BATCH, HIDDEN), jnp.float32),
)


@jax.jit
def kernel(segment_features, lane_table, type_table, length_table, W, b):
    idx = segment_features.astype(jnp.int32)
    e0T, e1T, e2T = _make_gather()(
        idx[:, 0], idx[:, 1], idx[:, 2],
        lane_table.T, type_table.T, length_table.T)
    Wt = W.T
    return _matmul(e0T, e1T, e2T,
                   Wt[:D0], Wt[D0:D0 + D1], Wt[D0 + D1:],
                   b.reshape(1, HIDDEN))
